# Initial kernel scaffold; baseline (speedup 1.0000x reference)
import jax
import jax.numpy as jnp
from jax.experimental import pallas as pl


def kernel(x, table):
    raise NotImplementedError("write your pallas kernel here")


# speedup vs baseline: 1.8771x; 1.8771x over previous
import functools

import jax
import jax.numpy as jnp
from jax import lax
from jax.experimental import pallas as pl
from jax.experimental.pallas import tpu as pltpu
from jax.experimental.pallas import tpu_sc as plsc

NC = 2
NS = 16
NW = NC * NS

G = 128
NBUF = 4


def _emb_lookup(idx_grouped, table, num_rows, num_groups):
    d = table.shape[1]
    rows_per_w = num_groups * G
    mesh = plsc.VectorSubcoreMesh(core_axis_name="c", subcore_axis_name="s")

    @functools.partial(
        pl.kernel,
        out_type=jax.ShapeDtypeStruct((num_rows, d), jnp.float32),
        mesh=mesh,
        scratch_types=[
            pltpu.VMEM((num_groups, G), jnp.int32),
            pltpu.VMEM((G, d), jnp.float32),
            pltpu.VMEM((G, d), jnp.float32),
            pltpu.VMEM((G, d), jnp.float32),
            pltpu.VMEM((G, d), jnp.float32),
            pltpu.SemaphoreType.DMA,
            pltpu.SemaphoreType.DMA,
            pltpu.SemaphoreType.DMA,
            pltpu.SemaphoreType.DMA,
        ],
        compiler_params=pltpu.CompilerParams(use_tc_tiling_on_sc=False),
    )
    def k(idx_hbm, table_hbm, out_hbm, idx_v, b0, b1, b2, b3, s0, s1, s2, s3):
        bufs = (b0, b1, b2, b3)
        sems = (s0, s1, s2, s3)
        wid = lax.axis_index("s") * NC + lax.axis_index("c")
        base = wid * rows_per_w

        pltpu.sync_copy(idx_hbm.at[wid], idx_v)

        for b in range(NBUF):
            pltpu.async_copy(table_hbm.at[idx_v.at[b]], bufs[b], sems[b])

        @pl.loop(0, num_groups, step=NBUF)
        def _(j):
            for b in range(NBUF):
                jj = j + b
                pltpu.make_async_copy(
                    table_hbm.at[idx_v.at[jj]], bufs[b], sems[b]
                ).wait()
                pltpu.sync_copy(bufs[b], out_hbm.at[pl.ds(base + jj * G, G)])

                @pl.when(jj + NBUF < num_groups)
                def _():
                    pltpu.async_copy(
                        table_hbm.at[idx_v.at[jj + NBUF]], bufs[b], sems[b]
                    )

    return k(idx_grouped, table)


def kernel(x, table):
    batch, hist = x.shape
    d = table.shape[1]
    num_rows = batch * hist
    num_groups = num_rows // (NW * G)
    idx = x.astype(jnp.int32).reshape(NW, num_groups, G)
    out = _emb_lookup(idx, table, num_rows, num_groups)
    return out.reshape(batch, hist, d)
